# baseline (speedup 1.0000x reference)
import functools

import jax
import jax.numpy as jnp
from jax import lax
from jax.experimental import pallas as pl
from jax.experimental.pallas import tpu as pltpu
from jax.experimental.pallas import tpu_sc as plsc

N0, N1, N2 = 100000, 50000, 25000
E1, E2 = 500000, 250000
D = 128
NB = 1000
EPS = 1e-5

B = 1024
K = 64
KSH, KMSK = 6, 63
NBUF = 4
NTILES = 16
NSC = 2

EPAD1, NPAD1 = 524288, 56832
EPAD2, NPAD2 = 262144, 25600


def _make_sc_agg(n_pass, nbpt, ch):
    cpt = ch // NTILES
    nfull, rem = cpt // K, cpt % K
    cbuf = B + K + 16
    npad = n_pass * NSC * ch
    mesh = plsc.VectorSubcoreMesh(core_axis_name="c", subcore_axis_name="s",
                                  num_cores=NSC, num_subcores=NTILES)

    @functools.partial(
        pl.kernel,
        mesh=mesh,
        out_type=jax.ShapeDtypeStruct((npad, D), jnp.float32),
        scratch_types=[
            pltpu.VMEM((B,), jnp.int32),
            pltpu.VMEM((B,), jnp.int32),
            pltpu.VMEM((cbuf,), jnp.int32),
            pltpu.VMEM((B // K + 1, K), jnp.int32),
            pltpu.VMEM((NBUF, K, D), jnp.float32),
            pltpu.VMEM_SHARED((ch + 16, D), jnp.float32),
            pltpu.SemaphoreType.DMA((NBUF,)),
            pltpu.SemaphoreType.DMA((NBUF,)),
        ],
        compiler_params=pltpu.CompilerParams(needs_layout_passes=False),
    )
    def agg(table, edges, out, dstb, srcb, csrc, cdst, rows, acc, gsems, ssems):
        c = lax.axis_index("c")
        s = lax.axis_index("s")
        r0 = s * cpt
        iota16 = lax.broadcasted_iota(jnp.int32, (16,), 0)

        for p in range(n_pass):
            lo = (p * NSC + c) * ch
            hi = lo + ch

            def zbody(i, _):
                rows[0, i // 8, pl.ds((i % 8) * 16, 16)] = jnp.zeros((16,), jnp.float32)
                return 0
            lax.fori_loop(0, K * 8, zbody, 0)
            for j in range(nfull):
                pltpu.sync_copy(rows.at[0], acc.at[pl.ds(r0 + j * K, K)])
            if rem:
                pltpu.sync_copy(rows.at[0, pl.ds(0, rem)],
                                acc.at[pl.ds(r0 + nfull * K, rem)])

            @pl.when(s == 0)
            def _():
                pltpu.sync_copy(rows.at[0, pl.ds(0, 16)], acc.at[pl.ds(ch, 16)])

            plsc.subcore_barrier()

            def fbody(it, _):
                off = (it * NTILES + s) * B
                pltpu.sync_copy(edges.at[0, pl.ds(off, B)], dstb)
                pltpu.sync_copy(edges.at[1, pl.ds(off, B)], srcb)

                def vbody(v, cnt):
                    dv = dstb[pl.ds(v * 16, 16)]
                    sv = srcb[pl.ds(v * 16, 16)]
                    m = (dv >= lo) & (dv < hi)
                    x = jnp.where(m, 1, 0)
                    for kk in (1, 2, 4, 8):
                        sh = x.at[jnp.maximum(iota16 - kk, 0)].get(
                            mode="promise_in_bounds")
                        x = x + jnp.where(iota16 >= kk, sh, 0)
                    pos = cnt + x - 1
                    plsc.store_scatter(csrc, [pos], sv, mask=m)
                    plsc.store_scatter(
                        cdst, [lax.shift_right_logical(pos, KSH),
                               lax.bitwise_and(pos, KMSK)], dv - lo, mask=m)
                    return cnt + x[15]

                cnt = lax.fori_loop(0, B // 16, vbody, jnp.int32(0))

                for j in range(K // 16):
                    ppos = cnt + j * 16 + iota16
                    plsc.store_scatter(csrc, [ppos], jnp.zeros((16,), jnp.int32))
                    plsc.store_scatter(
                        cdst, [lax.shift_right_logical(ppos, KSH),
                               lax.bitwise_and(ppos, KMSK)],
                        jnp.full((16,), ch, jnp.int32))
                nblk = (cnt + K - 1) // K

                for j in range(2):
                    @pl.when(j < nblk)
                    def _(j=j):
                        pltpu.async_copy(table.at[csrc.at[pl.ds(j * K, K)]],
                                         rows.at[j], gsems.at[j])

                def gbody(i, _):
                    b = lax.rem(i, NBUF)
                    pltpu.make_async_copy(
                        table.at[csrc.at[pl.ds(i * K, K)]],
                        rows.at[b], gsems.at[b]).wait()
                    pltpu.async_copy(rows.at[b], acc.at[cdst.at[i]],
                                     ssems.at[b], add=True)

                    @pl.when(i + 2 < nblk)
                    def _():
                        b2 = lax.rem(i + 2, NBUF)

                        @pl.when(i - 2 >= 0)
                        def _():
                            pltpu.make_async_copy(
                                rows.at[b2], acc.at[cdst.at[i - 2]],
                                ssems.at[b2]).wait()
                        pltpu.async_copy(
                            table.at[csrc.at[pl.ds((i + 2) * K, K)]],
                            rows.at[b2], gsems.at[b2])
                    return 0

                lax.fori_loop(0, nblk, gbody, 0)

                def dbody(i, _):
                    b = lax.rem(i, NBUF)
                    pltpu.make_async_copy(rows.at[b], acc.at[cdst.at[i]],
                                          ssems.at[b]).wait()
                    return 0
                lax.fori_loop(jnp.maximum(nblk - 4, 0), nblk, dbody, 0)
                return 0

            lax.fori_loop(0, nbpt, fbody, jnp.int32(0))
            plsc.subcore_barrier()

            for j in range(nfull):
                pltpu.sync_copy(acc.at[pl.ds(r0 + j * K, K)], rows.at[0])
                pltpu.sync_copy(rows.at[0], out.at[pl.ds(lo + r0 + j * K, K)])
            if rem:
                pltpu.sync_copy(acc.at[pl.ds(r0 + nfull * K, rem)],
                                rows.at[0, pl.ds(0, rem)])
                pltpu.sync_copy(rows.at[0, pl.ds(0, rem)],
                                out.at[pl.ds(lo + r0 + nfull * K, rem)])

    return agg


@functools.lru_cache(maxsize=None)
def _agg1():
    return _make_sc_agg(n_pass=3, nbpt=32, ch=9472)


@functools.lru_cache(maxsize=None)
def _agg2():
    return _make_sc_agg(n_pass=2, nbpt=16, ch=6400)


def _mlp_pool(h, wa, ba, wb, bb, ids3, rblk):
    n = h.shape[0]
    grid = n // rblk

    def body(h_ref, wa_ref, ba_ref, wb_ref, bb_ref, ids_ref, out_ref, pool_ref):
        i = pl.program_id(0)
        a = jnp.maximum(
            jnp.dot(h_ref[...], wa_ref[...], preferred_element_type=jnp.float32)
            + ba_ref[...], 0.0)
        y = jnp.maximum(
            jnp.dot(a, wb_ref[...], preferred_element_type=jnp.float32)
            + bb_ref[...], 0.0)
        out_ref[...] = y
        ids = ids_ref[0, 0, :]
        oh = (ids[:, None] == lax.broadcasted_iota(jnp.int32, (rblk, NB), 1)
              ).astype(jnp.bfloat16)
        part = lax.dot_general(oh, y.astype(jnp.bfloat16),
                               (((0,), (0,)), ((), ())),
                               preferred_element_type=jnp.float32)

        @pl.when(i == 0)
        def _():
            pool_ref[...] = jnp.zeros_like(pool_ref)

        pool_ref[...] += part

    return pl.pallas_call(
        body,
        grid=(grid,),
        in_specs=[
            pl.BlockSpec((rblk, D), lambda i: (i, 0)),
            pl.BlockSpec((D, D), lambda i: (0, 0)),
            pl.BlockSpec((1, D), lambda i: (0, 0)),
            pl.BlockSpec((D, D), lambda i: (0, 0)),
            pl.BlockSpec((1, D), lambda i: (0, 0)),
            pl.BlockSpec((1, 1, rblk), lambda i: (i, 0, 0)),
        ],
        out_specs=[
            pl.BlockSpec((rblk, D), lambda i: (i, 0)),
            pl.BlockSpec((NB, D), lambda i: (0, 0)),
        ],
        out_shape=[
            jax.ShapeDtypeStruct((n, D), jnp.float32),
            jax.ShapeDtypeStruct((NB, D), jnp.float32),
        ],
    )(h, wa, ba, wb, bb, ids3)


def _pool_only(x, ids3, rblk):
    n = x.shape[0]
    grid = n // rblk

    def body(x_ref, ids_ref, pool_ref):
        i = pl.program_id(0)
        ids = ids_ref[0, 0, :]
        oh = (ids[:, None] == lax.broadcasted_iota(jnp.int32, (rblk, NB), 1)
              ).astype(jnp.bfloat16)
        part = lax.dot_general(oh, x_ref[...].astype(jnp.bfloat16),
                               (((0,), (0,)), ((), ())),
                               preferred_element_type=jnp.float32)

        @pl.when(i == 0)
        def _():
            pool_ref[...] = jnp.zeros_like(pool_ref)

        pool_ref[...] += part

    return pl.pallas_call(
        body,
        grid=(grid,),
        in_specs=[
            pl.BlockSpec((rblk, D), lambda i: (i, 0)),
            pl.BlockSpec((1, 1, rblk), lambda i: (i, 0, 0)),
        ],
        out_specs=pl.BlockSpec((NB, D), lambda i: (0, 0)),
        out_shape=jax.ShapeDtypeStruct((NB, D), jnp.float32),
    )(x, ids3)


def _combine(p0, p1, p2, wd, bd):
    def body(p0_ref, p1_ref, p2_ref, wd_ref, bd_ref, out_ref):
        out_ref[...] = (
            jnp.dot(p0_ref[...], wd_ref[0:D, :], preferred_element_type=jnp.float32)
            + jnp.dot(p1_ref[...], wd_ref[D:2 * D, :], preferred_element_type=jnp.float32)
            + jnp.dot(p2_ref[...], wd_ref[2 * D:3 * D, :], preferred_element_type=jnp.float32)
            + bd_ref[...])

    return pl.pallas_call(
        body,
        out_shape=jax.ShapeDtypeStruct((NB, D), jnp.float32),
    )(p0, p1, p2, wd, bd)


def _pad_edges(e, e_pad, n_edges):
    fill = jnp.broadcast_to(jnp.array([[-1], [0]], jnp.int32), (2, e_pad - n_edges))
    return jnp.concatenate([e, fill], axis=1)


def _pad_ids(ids, npad, rblk):
    n = ids.shape[0]
    if npad > n:
        ids = jnp.concatenate([ids, jnp.full((npad - n,), NB, jnp.int32)])
    return ids.reshape(npad // rblk, 1, rblk)


def kernel(x, edge_index1, edge_index2, batch0, batch1, batch2,
           W1a, b1a, g1a, be1a, W1b, b1b, g1b, be1b,
           W2a, b2a, W2b, b2b, Wd, bd):
    s = 1.0 / jnp.sqrt(1.0 + EPS)
    W1a_f = W1a * (g1a * s)[None, :]
    b1a_f = (b1a * (g1a * s) + be1a)[None, :]
    W1b_f = W1b * (g1b * s)[None, :]
    b1b_f = (b1b * (g1b * s) + be1b)[None, :]

    e1p = _pad_edges(edge_index1, EPAD1, E1)
    e2p = _pad_edges(edge_index2, EPAD2, E2)

    p0 = _pool_only(x, _pad_ids(batch0, 100000, 800), 800)
    agg1 = _agg1()(x, e1p)
    x1p, p1 = _mlp_pool(agg1, W1a_f, b1a_f, W1b_f, b1b_f,
                        _pad_ids(batch1, NPAD1, 512), 512)
    agg2 = _agg2()(x1p, e2p)
    _, p2 = _mlp_pool(agg2, W2a, b2a[None, :], W2b, b2b[None, :],
                      _pad_ids(batch2, NPAD2, 512), 512)
    return _combine(p0, p1, p2, Wd, bd[None, :])


# speedup vs baseline: 2.8153x; 2.8153x over previous
import functools

import jax
import jax.numpy as jnp
from jax import lax
from jax.experimental import pallas as pl
from jax.experimental.pallas import tpu as pltpu
from jax.experimental.pallas import tpu_sc as plsc

N0, N1, N2 = 100000, 50000, 25000
E1, E2 = 500000, 250000
D = 128
NB = 1000
EPS = 1e-5

B = 4096
K = 64
KSH, KMSK = 6, 63
NTILES = 16
NSC = 2

EPAD1, NPAD1 = 524288, 59904
EPAD2, NPAD2 = 262144, 25600


def _make_sc_agg(n_pass, nbpt, ch):
    cpt = ch // NTILES
    nfull, rem = cpt // K, cpt % K
    cbuf = B + K + 16
    npad = n_pass * NSC * ch
    mesh = plsc.VectorSubcoreMesh(core_axis_name="c", subcore_axis_name="s",
                                  num_cores=NSC, num_subcores=NTILES)

    @functools.partial(
        pl.kernel,
        mesh=mesh,
        out_type=jax.ShapeDtypeStruct((npad, D), jnp.float32),
        scratch_types=[
            pltpu.VMEM((B,), jnp.int32),
            pltpu.VMEM((B,), jnp.int32),
            pltpu.VMEM((cbuf,), jnp.int32),
            pltpu.VMEM((B // K + 1, K), jnp.int32),
            pltpu.VMEM((2, K, D), jnp.float32),
            pltpu.VMEM_SHARED((ch + 16, D), jnp.float32),
            pltpu.SemaphoreType.DMA((2,)),
        ],
        compiler_params=pltpu.CompilerParams(needs_layout_passes=False),
    )
    def agg(table, edges, out, dstb, srcb, csrc, cdst, rows, acc, sems):
        c = lax.axis_index("c")
        s = lax.axis_index("s")
        r0 = s * cpt
        iota16 = lax.broadcasted_iota(jnp.int32, (16,), 0)

        for p in range(n_pass):
            lo = (p * NSC + c) * ch
            hi = lo + ch

            def zbody(i, _):
                rows[0, i // 8, pl.ds((i % 8) * 16, 16)] = jnp.zeros((16,), jnp.float32)
                return 0
            lax.fori_loop(0, K * 8, zbody, 0)
            for j in range(nfull):
                pltpu.sync_copy(rows.at[0], acc.at[pl.ds(r0 + j * K, K)])
            if rem:
                pltpu.sync_copy(rows.at[0, pl.ds(0, rem)],
                                acc.at[pl.ds(r0 + nfull * K, rem)])

            @pl.when(s == 0)
            def _():
                pltpu.sync_copy(rows.at[0, pl.ds(0, 16)], acc.at[pl.ds(ch, 16)])

            plsc.subcore_barrier()

            def fbody(it, _):
                off = (it * NTILES + s) * B
                pltpu.sync_copy(edges.at[0, pl.ds(off, B)], dstb)
                pltpu.sync_copy(edges.at[1, pl.ds(off, B)], srcb)

                def vbody(v, cnt):
                    dv = dstb[pl.ds(v * 16, 16)]
                    sv = srcb[pl.ds(v * 16, 16)]
                    m = (dv >= lo) & (dv < hi)
                    x = jnp.where(m, 1, 0)
                    for kk in (1, 2, 4, 8):
                        sh = x.at[jnp.maximum(iota16 - kk, 0)].get(
                            mode="promise_in_bounds")
                        x = x + jnp.where(iota16 >= kk, sh, 0)
                    pos = cnt + x - 1
                    plsc.store_scatter(csrc, [pos], sv, mask=m)
                    plsc.store_scatter(
                        cdst, [lax.shift_right_logical(pos, KSH),
                               lax.bitwise_and(pos, KMSK)], dv - lo, mask=m)
                    return cnt + x[15]

                cnt = lax.fori_loop(0, B // 16, vbody, jnp.int32(0))

                for j in range(K // 16):
                    ppos = cnt + j * 16 + iota16
                    plsc.store_scatter(csrc, [ppos], jnp.zeros((16,), jnp.int32))
                    plsc.store_scatter(
                        cdst, [lax.shift_right_logical(ppos, KSH),
                               lax.bitwise_and(ppos, KMSK)],
                        jnp.full((16,), ch, jnp.int32))
                nblk = (cnt + K - 1) // K

                @pl.when(nblk > 0)
                def _():
                    pltpu.async_copy(table.at[csrc.at[pl.ds(0, K)]],
                                     rows.at[0], sems.at[0])

                def gbody(i, _):
                    par = lax.rem(i, 2)
                    nxt = lax.rem(i + 1, 2)

                    @pl.when(i + 1 < nblk)
                    def _():
                        pltpu.async_copy(
                            table.at[csrc.at[pl.ds((i + 1) * K, K)]],
                            rows.at[nxt], sems.at[nxt])

                    pltpu.make_async_copy(
                        table.at[csrc.at[pl.ds(i * K, K)]],
                        rows.at[par], sems.at[par]).wait()
                    pltpu.sync_copy(rows.at[par], acc.at[cdst.at[i]], add=True)
                    return 0

                lax.fori_loop(0, nblk, gbody, 0)
                return 0

            lax.fori_loop(0, nbpt, fbody, jnp.int32(0))
            plsc.subcore_barrier()

            for j in range(nfull):
                pltpu.sync_copy(acc.at[pl.ds(r0 + j * K, K)], rows.at[0])
                pltpu.sync_copy(rows.at[0], out.at[pl.ds(lo + r0 + j * K, K)])
            if rem:
                pltpu.sync_copy(acc.at[pl.ds(r0 + nfull * K, rem)],
                                rows.at[0, pl.ds(0, rem)])
                pltpu.sync_copy(rows.at[0, pl.ds(0, rem)],
                                out.at[pl.ds(lo + r0 + nfull * K, rem)])

    return agg


@functools.lru_cache(maxsize=None)
def _agg1():
    return _make_sc_agg(n_pass=3, nbpt=8, ch=9984)


@functools.lru_cache(maxsize=None)
def _agg2():
    return _make_sc_agg(n_pass=2, nbpt=4, ch=6400)


def _mlp_pool(h, wa, ba, wb, bb, ids3, rblk):
    n = h.shape[0]
    grid = n // rblk

    def body(h_ref, wa_ref, ba_ref, wb_ref, bb_ref, ids_ref, out_ref, pool_ref):
        i = pl.program_id(0)
        a = jnp.maximum(
            jnp.dot(h_ref[...], wa_ref[...], preferred_element_type=jnp.float32)
            + ba_ref[...], 0.0)
        y = jnp.maximum(
            jnp.dot(a, wb_ref[...], preferred_element_type=jnp.float32)
            + bb_ref[...], 0.0)
        out_ref[...] = y
        ids = ids_ref[0, 0, :]
        oh = (ids[:, None] == lax.broadcasted_iota(jnp.int32, (rblk, NB), 1)
              ).astype(jnp.bfloat16)
        part = lax.dot_general(oh, y.astype(jnp.bfloat16),
                               (((0,), (0,)), ((), ())),
                               preferred_element_type=jnp.float32)

        @pl.when(i == 0)
        def _():
            pool_ref[...] = jnp.zeros_like(pool_ref)

        pool_ref[...] += part

    return pl.pallas_call(
        body,
        grid=(grid,),
        in_specs=[
            pl.BlockSpec((rblk, D), lambda i: (i, 0)),
            pl.BlockSpec((D, D), lambda i: (0, 0)),
            pl.BlockSpec((1, D), lambda i: (0, 0)),
            pl.BlockSpec((D, D), lambda i: (0, 0)),
            pl.BlockSpec((1, D), lambda i: (0, 0)),
            pl.BlockSpec((1, 1, rblk), lambda i: (i, 0, 0)),
        ],
        out_specs=[
            pl.BlockSpec((rblk, D), lambda i: (i, 0)),
            pl.BlockSpec((NB, D), lambda i: (0, 0)),
        ],
        out_shape=[
            jax.ShapeDtypeStruct((n, D), jnp.float32),
            jax.ShapeDtypeStruct((NB, D), jnp.float32),
        ],
    )(h, wa, ba, wb, bb, ids3)


def _pool_only(x, ids3, rblk):
    n = x.shape[0]
    grid = n // rblk

    def body(x_ref, ids_ref, pool_ref):
        i = pl.program_id(0)
        ids = ids_ref[0, 0, :]
        oh = (ids[:, None] == lax.broadcasted_iota(jnp.int32, (rblk, NB), 1)
              ).astype(jnp.bfloat16)
        part = lax.dot_general(oh, x_ref[...].astype(jnp.bfloat16),
                               (((0,), (0,)), ((), ())),
                               preferred_element_type=jnp.float32)

        @pl.when(i == 0)
        def _():
            pool_ref[...] = jnp.zeros_like(pool_ref)

        pool_ref[...] += part

    return pl.pallas_call(
        body,
        grid=(grid,),
        in_specs=[
            pl.BlockSpec((rblk, D), lambda i: (i, 0)),
            pl.BlockSpec((1, 1, rblk), lambda i: (i, 0, 0)),
        ],
        out_specs=pl.BlockSpec((NB, D), lambda i: (0, 0)),
        out_shape=jax.ShapeDtypeStruct((NB, D), jnp.float32),
    )(x, ids3)


def _combine(p0, p1, p2, wd, bd):
    def body(p0_ref, p1_ref, p2_ref, wd_ref, bd_ref, out_ref):
        out_ref[...] = (
            jnp.dot(p0_ref[...], wd_ref[0:D, :], preferred_element_type=jnp.float32)
            + jnp.dot(p1_ref[...], wd_ref[D:2 * D, :], preferred_element_type=jnp.float32)
            + jnp.dot(p2_ref[...], wd_ref[2 * D:3 * D, :], preferred_element_type=jnp.float32)
            + bd_ref[...])

    return pl.pallas_call(
        body,
        out_shape=jax.ShapeDtypeStruct((NB, D), jnp.float32),
    )(p0, p1, p2, wd, bd)


def _pad_edges(e, e_pad, n_edges):
    fill = jnp.broadcast_to(jnp.array([[-1], [0]], jnp.int32), (2, e_pad - n_edges))
    return jnp.concatenate([e, fill], axis=1)


def _pad_ids(ids, npad, rblk):
    n = ids.shape[0]
    if npad > n:
        ids = jnp.concatenate([ids, jnp.full((npad - n,), NB, jnp.int32)])
    return ids.reshape(npad // rblk, 1, rblk)


def kernel(x, edge_index1, edge_index2, batch0, batch1, batch2,
           W1a, b1a, g1a, be1a, W1b, b1b, g1b, be1b,
           W2a, b2a, W2b, b2b, Wd, bd):
    s = 1.0 / jnp.sqrt(1.0 + EPS)
    W1a_f = W1a * (g1a * s)[None, :]
    b1a_f = (b1a * (g1a * s) + be1a)[None, :]
    W1b_f = W1b * (g1b * s)[None, :]
    b1b_f = (b1b * (g1b * s) + be1b)[None, :]

    e1p = _pad_edges(edge_index1, EPAD1, E1)
    e2p = _pad_edges(edge_index2, EPAD2, E2)

    p0 = _pool_only(x, _pad_ids(batch0, 100000, 800), 800)
    agg1 = _agg1()(x, e1p)
    x1p, p1 = _mlp_pool(agg1, W1a_f, b1a_f, W1b_f, b1b_f,
                        _pad_ids(batch1, NPAD1, 512), 512)
    agg2 = _agg2()(x1p, e2p)
    _, p2 = _mlp_pool(agg2, W2a, b2a[None, :], W2b, b2b[None, :],
                      _pad_ids(batch2, NPAD2, 512), 512)
    return _combine(p0, p1, p2, Wd, bd[None, :])


# speedup vs baseline: 3.9540x; 1.4045x over previous
import functools

import jax
import jax.numpy as jnp
from jax import lax
from jax.experimental import pallas as pl
from jax.experimental.pallas import tpu as pltpu
from jax.experimental.pallas import tpu_sc as plsc

N0, N1, N2 = 100000, 50000, 25000
E1, E2 = 500000, 250000
D = 128
NB = 1000
EPS = 1e-5

B = 8192
K = 64
KSH, KMSK = 6, 63
NTILES = 16
NSC = 2

EPAD1, NPAD1 = 524288, 51200
EPAD2, NPAD2 = 262144, 25600


def _make_sc_agg(n_pass, nbpt, ch):
    cpt = ch // NTILES
    nfull, rem = cpt // K, cpt % K
    cbuf = B + K + 16
    npad = n_pass * NSC * ch
    mesh = plsc.VectorSubcoreMesh(core_axis_name="c", subcore_axis_name="s",
                                  num_cores=NSC, num_subcores=NTILES)

    @functools.partial(
        pl.kernel,
        mesh=mesh,
        out_type=jax.ShapeDtypeStruct((npad, D), jnp.float32),
        scratch_types=[
            pltpu.VMEM((2, B), jnp.int32),
            pltpu.VMEM((cbuf,), jnp.int32),
            pltpu.VMEM((B // K + 1, K), jnp.int32),
            pltpu.VMEM((2, K, D), jnp.float32),
            pltpu.VMEM_SHARED((ch + 16, D), jnp.float32),
            pltpu.SemaphoreType.DMA((2,)),
        ],
        compiler_params=pltpu.CompilerParams(needs_layout_passes=False),
    )
    def agg(table, edges, out, ebuf, csrc, cdst, rows, acc, sems):
        c = lax.axis_index("c")
        s = lax.axis_index("s")
        r0 = s * cpt
        iota16 = lax.broadcasted_iota(jnp.int32, (16,), 0)

        for p in range(n_pass):
            lo = (p * NSC + c) * ch
            hi = lo + ch

            def zbody(i, _):
                rows[0, i // 8, pl.ds((i % 8) * 16, 16)] = jnp.zeros((16,), jnp.float32)
                return 0
            lax.fori_loop(0, K * 8, zbody, 0)
            for j in range(nfull):
                pltpu.sync_copy(rows.at[0], acc.at[pl.ds(r0 + j * K, K)])
            if rem:
                pltpu.sync_copy(rows.at[0, pl.ds(0, rem)],
                                acc.at[pl.ds(r0 + nfull * K, rem)])

            @pl.when(s == 0)
            def _():
                pltpu.sync_copy(rows.at[0, pl.ds(0, 16)], acc.at[pl.ds(ch, 16)])

            plsc.subcore_barrier()

            def fbody(it, _):
                off = (it * NTILES + s) * B
                pltpu.sync_copy(edges.at[:, pl.ds(off, B)], ebuf)

                def vbody(v, cnt):
                    dv = ebuf[0, pl.ds(v * 16, 16)]
                    sv = ebuf[1, pl.ds(v * 16, 16)]
                    m = (dv >= lo) & (dv < hi)
                    x = jnp.where(m, 1, 0)
                    for kk in (1, 2, 4, 8):
                        sh = x.at[jnp.maximum(iota16 - kk, 0)].get(
                            mode="promise_in_bounds")
                        x = x + jnp.where(iota16 >= kk, sh, 0)
                    pos = cnt + x - 1
                    plsc.store_scatter(csrc, [pos], sv, mask=m)
                    plsc.store_scatter(
                        cdst, [lax.shift_right_logical(pos, KSH),
                               lax.bitwise_and(pos, KMSK)], dv - lo, mask=m)
                    return cnt + x[15]

                cnt = lax.fori_loop(0, B // 16, vbody, jnp.int32(0))

                for j in range(K // 16):
                    ppos = cnt + j * 16 + iota16
                    plsc.store_scatter(csrc, [ppos], jnp.zeros((16,), jnp.int32))
                    plsc.store_scatter(
                        cdst, [lax.shift_right_logical(ppos, KSH),
                               lax.bitwise_and(ppos, KMSK)],
                        jnp.full((16,), ch, jnp.int32))
                nblk = (cnt + K - 1) // K

                @pl.when(nblk > 0)
                def _():
                    pltpu.async_copy(table.at[csrc.at[pl.ds(0, K)]],
                                     rows.at[0], sems.at[0])

                def gbody(i, _):
                    par = lax.rem(i, 2)
                    nxt = lax.rem(i + 1, 2)

                    @pl.when(i + 1 < nblk)
                    def _():
                        pltpu.async_copy(
                            table.at[csrc.at[pl.ds((i + 1) * K, K)]],
                            rows.at[nxt], sems.at[nxt])

                    pltpu.make_async_copy(
                        table.at[csrc.at[pl.ds(i * K, K)]],
                        rows.at[par], sems.at[par]).wait()
                    pltpu.sync_copy(rows.at[par], acc.at[cdst.at[i]], add=True)
                    return 0

                lax.fori_loop(0, nblk, gbody, 0)
                return 0

            lax.fori_loop(0, nbpt, fbody, jnp.int32(0))
            plsc.subcore_barrier()

            for j in range(nfull):
                pltpu.sync_copy(acc.at[pl.ds(r0 + j * K, K)], rows.at[0])
                pltpu.sync_copy(rows.at[0], out.at[pl.ds(lo + r0 + j * K, K)])
            if rem:
                pltpu.sync_copy(acc.at[pl.ds(r0 + nfull * K, rem)],
                                rows.at[0, pl.ds(0, rem)])
                pltpu.sync_copy(rows.at[0, pl.ds(0, rem)],
                                out.at[pl.ds(lo + r0 + nfull * K, rem)])

    return agg


@functools.lru_cache(maxsize=None)
def _agg1():
    return _make_sc_agg(n_pass=4, nbpt=4, ch=6400)


@functools.lru_cache(maxsize=None)
def _agg2():
    return _make_sc_agg(n_pass=2, nbpt=2, ch=6400)


def _mlp_pool(h, wa, ba, wb, bb, ids3, rblk):
    n = h.shape[0]
    grid = n // rblk

    def body(h_ref, wa_ref, ba_ref, wb_ref, bb_ref, ids_ref, out_ref, pool_ref):
        i = pl.program_id(0)
        a = jnp.maximum(
            jnp.dot(h_ref[...], wa_ref[...], preferred_element_type=jnp.float32)
            + ba_ref[...], 0.0)
        y = jnp.maximum(
            jnp.dot(a, wb_ref[...], preferred_element_type=jnp.float32)
            + bb_ref[...], 0.0)
        out_ref[...] = y
        ids = ids_ref[0, 0, :]
        oh = (ids[:, None] == lax.broadcasted_iota(jnp.int32, (rblk, NB), 1)
              ).astype(jnp.bfloat16)
        part = lax.dot_general(oh, y.astype(jnp.bfloat16),
                               (((0,), (0,)), ((), ())),
                               preferred_element_type=jnp.float32)

        @pl.when(i == 0)
        def _():
            pool_ref[...] = jnp.zeros_like(pool_ref)

        pool_ref[...] += part

    return pl.pallas_call(
        body,
        grid=(grid,),
        in_specs=[
            pl.BlockSpec((rblk, D), lambda i: (i, 0)),
            pl.BlockSpec((D, D), lambda i: (0, 0)),
            pl.BlockSpec((1, D), lambda i: (0, 0)),
            pl.BlockSpec((D, D), lambda i: (0, 0)),
            pl.BlockSpec((1, D), lambda i: (0, 0)),
            pl.BlockSpec((1, 1, rblk), lambda i: (i, 0, 0)),
        ],
        out_specs=[
            pl.BlockSpec((rblk, D), lambda i: (i, 0)),
            pl.BlockSpec((NB, D), lambda i: (0, 0)),
        ],
        out_shape=[
            jax.ShapeDtypeStruct((n, D), jnp.float32),
            jax.ShapeDtypeStruct((NB, D), jnp.float32),
        ],
    )(h, wa, ba, wb, bb, ids3)


def _pool_only(x, ids3, rblk):
    n = x.shape[0]
    grid = n // rblk

    def body(x_ref, ids_ref, pool_ref):
        i = pl.program_id(0)
        ids = ids_ref[0, 0, :]
        oh = (ids[:, None] == lax.broadcasted_iota(jnp.int32, (rblk, NB), 1)
              ).astype(jnp.bfloat16)
        part = lax.dot_general(oh, x_ref[...].astype(jnp.bfloat16),
                               (((0,), (0,)), ((), ())),
                               preferred_element_type=jnp.float32)

        @pl.when(i == 0)
        def _():
            pool_ref[...] = jnp.zeros_like(pool_ref)

        pool_ref[...] += part

    return pl.pallas_call(
        body,
        grid=(grid,),
        in_specs=[
            pl.BlockSpec((rblk, D), lambda i: (i, 0)),
            pl.BlockSpec((1, 1, rblk), lambda i: (i, 0, 0)),
        ],
        out_specs=pl.BlockSpec((NB, D), lambda i: (0, 0)),
        out_shape=jax.ShapeDtypeStruct((NB, D), jnp.float32),
    )(x, ids3)


def _combine(p0, p1, p2, wd, bd):
    def body(p0_ref, p1_ref, p2_ref, wd_ref, bd_ref, out_ref):
        out_ref[...] = (
            jnp.dot(p0_ref[...], wd_ref[0:D, :], preferred_element_type=jnp.float32)
            + jnp.dot(p1_ref[...], wd_ref[D:2 * D, :], preferred_element_type=jnp.float32)
            + jnp.dot(p2_ref[...], wd_ref[2 * D:3 * D, :], preferred_element_type=jnp.float32)
            + bd_ref[...])

    return pl.pallas_call(
        body,
        out_shape=jax.ShapeDtypeStruct((NB, D), jnp.float32),
    )(p0, p1, p2, wd, bd)


def _pad_edges(e, e_pad, n_edges):
    fill = jnp.broadcast_to(jnp.array([[-1], [0]], jnp.int32), (2, e_pad - n_edges))
    return jnp.concatenate([e, fill], axis=1)


def _pad_ids(ids, npad, rblk):
    n = ids.shape[0]
    if npad > n:
        ids = jnp.concatenate([ids, jnp.full((npad - n,), NB, jnp.int32)])
    return ids.reshape(npad // rblk, 1, rblk)


def kernel(x, edge_index1, edge_index2, batch0, batch1, batch2,
           W1a, b1a, g1a, be1a, W1b, b1b, g1b, be1b,
           W2a, b2a, W2b, b2b, Wd, bd):
    s = 1.0 / jnp.sqrt(1.0 + EPS)
    W1a_f = W1a * (g1a * s)[None, :]
    b1a_f = (b1a * (g1a * s) + be1a)[None, :]
    W1b_f = W1b * (g1b * s)[None, :]
    b1b_f = (b1b * (g1b * s) + be1b)[None, :]

    e1p = _pad_edges(edge_index1, EPAD1, E1)
    e2p = _pad_edges(edge_index2, EPAD2, E2)

    p0 = _pool_only(x, _pad_ids(batch0, 100000, 800), 800)
    agg1 = _agg1()(x, e1p)
    x1p, p1 = _mlp_pool(agg1, W1a_f, b1a_f, W1b_f, b1b_f,
                        _pad_ids(batch1, NPAD1, 512), 512)
    agg2 = _agg2()(x1p, e2p)
    _, p2 = _mlp_pool(agg2, W2a, b2a[None, :], W2b, b2b[None, :],
                      _pad_ids(batch2, NPAD2, 512), 512)
    return _combine(p0, p1, p2, Wd, bd[None, :])


# speedup vs baseline: 3.9574x; 1.0009x over previous
import functools

import jax
import jax.numpy as jnp
from jax import lax
from jax.experimental import pallas as pl
from jax.experimental.pallas import tpu as pltpu
from jax.experimental.pallas import tpu_sc as plsc

N0, N1, N2 = 100000, 50000, 25000
E1, E2 = 500000, 250000
D = 128
NB = 1000
EPS = 1e-5

B = 8192
K = 64
KSH, KMSK = 6, 63
NTILES = 16
NSC = 2

EPAD1, NPAD1 = 524288, 51200
EPAD2, NPAD2 = 262144, 25600


def _make_sc_agg(n_pass, nbpt, ch):
    cpt = ch // NTILES
    nfull, rem = cpt // K, cpt % K
    cbuf = B + K + 16
    npad = n_pass * NSC * ch
    mesh = plsc.VectorSubcoreMesh(core_axis_name="c", subcore_axis_name="s",
                                  num_cores=NSC, num_subcores=NTILES)

    @functools.partial(
        pl.kernel,
        mesh=mesh,
        out_type=jax.ShapeDtypeStruct((npad, D), jnp.float32),
        scratch_types=[
            pltpu.VMEM((2, B), jnp.int32),
            pltpu.VMEM((cbuf,), jnp.int32),
            pltpu.VMEM((B // K + 1, K), jnp.int32),
            pltpu.VMEM((2, K, D), jnp.float32),
            pltpu.VMEM_SHARED((ch + 16, D), jnp.float32),
            pltpu.SemaphoreType.DMA((2,)),
        ],
        compiler_params=pltpu.CompilerParams(needs_layout_passes=False),
    )
    def agg(table, edges, out, ebuf, csrc, cdst, rows, acc, sems):
        c = lax.axis_index("c")
        s = lax.axis_index("s")
        r0 = s * cpt
        iota16 = lax.broadcasted_iota(jnp.int32, (16,), 0)

        for p in range(n_pass):
            lo = (p * NSC + c) * ch
            hi = lo + ch

            def zbody(i, _):
                rows[0, i // 8, pl.ds((i % 8) * 16, 16)] = jnp.zeros((16,), jnp.float32)
                return 0
            lax.fori_loop(0, K * 8, zbody, 0)
            for j in range(nfull):
                pltpu.sync_copy(rows.at[0], acc.at[pl.ds(r0 + j * K, K)])
            if rem:
                pltpu.sync_copy(rows.at[0, pl.ds(0, rem)],
                                acc.at[pl.ds(r0 + nfull * K, rem)])

            @pl.when(s == 0)
            def _():
                pltpu.sync_copy(rows.at[0, pl.ds(0, 16)], acc.at[pl.ds(ch, 16)])

            plsc.subcore_barrier()

            def fbody(it, _):
                off = (it * NTILES + s) * B
                pltpu.sync_copy(edges.at[:, pl.ds(off, B)], ebuf)

                def vbody(v, cnt):
                    dv = ebuf[0, pl.ds(v * 16, 16)]
                    sv = ebuf[1, pl.ds(v * 16, 16)]
                    m = (dv >= lo) & (dv < hi)
                    x = jnp.where(m, 1, 0)
                    for kk in (1, 2, 4, 8):
                        sh = x.at[jnp.maximum(iota16 - kk, 0)].get(
                            mode="promise_in_bounds")
                        x = x + jnp.where(iota16 >= kk, sh, 0)
                    pos = cnt + x - 1
                    plsc.store_scatter(csrc, [pos], sv, mask=m)
                    plsc.store_scatter(
                        cdst, [lax.shift_right_logical(pos, KSH),
                               lax.bitwise_and(pos, KMSK)], dv - lo, mask=m)
                    return cnt + x[15]

                cnt = lax.fori_loop(0, B // 16, vbody, jnp.int32(0))

                for j in range(K // 16):
                    ppos = cnt + j * 16 + iota16
                    plsc.store_scatter(csrc, [ppos], jnp.zeros((16,), jnp.int32))
                    plsc.store_scatter(
                        cdst, [lax.shift_right_logical(ppos, KSH),
                               lax.bitwise_and(ppos, KMSK)],
                        jnp.full((16,), ch, jnp.int32))
                nblk = (cnt + K - 1) // K

                @pl.when(nblk > 0)
                def _():
                    pltpu.async_copy(table.at[csrc.at[pl.ds(0, K)]],
                                     rows.at[0], sems.at[0])

                def gbody(i, _):
                    par = lax.rem(i, 2)
                    nxt = lax.rem(i + 1, 2)

                    @pl.when(i + 1 < nblk)
                    def _():
                        pltpu.async_copy(
                            table.at[csrc.at[pl.ds((i + 1) * K, K)]],
                            rows.at[nxt], sems.at[nxt])

                    pltpu.make_async_copy(
                        table.at[csrc.at[pl.ds(i * K, K)]],
                        rows.at[par], sems.at[par]).wait()
                    pltpu.sync_copy(rows.at[par], acc.at[cdst.at[i]], add=True)
                    return 0

                lax.fori_loop(0, nblk, gbody, 0)
                return 0

            lax.fori_loop(0, nbpt, fbody, jnp.int32(0))
            plsc.subcore_barrier()

            for j in range(nfull):
                pltpu.sync_copy(acc.at[pl.ds(r0 + j * K, K)], rows.at[0])
                pltpu.sync_copy(rows.at[0], out.at[pl.ds(lo + r0 + j * K, K)])
            if rem:
                pltpu.sync_copy(acc.at[pl.ds(r0 + nfull * K, rem)],
                                rows.at[0, pl.ds(0, rem)])
                pltpu.sync_copy(rows.at[0, pl.ds(0, rem)],
                                out.at[pl.ds(lo + r0 + nfull * K, rem)])

    return agg


@functools.lru_cache(maxsize=None)
def _agg1():
    return _make_sc_agg(n_pass=4, nbpt=4, ch=6400)


@functools.lru_cache(maxsize=None)
def _agg2():
    return _make_sc_agg(n_pass=2, nbpt=2, ch=6400)


def _mlp_pool(h, wa, ba, wb, bb, ids3, rblk):
    n = h.shape[0]
    grid = n // rblk

    def body(h_ref, wa_ref, ba_ref, wb_ref, bb_ref, ids_ref, out_ref, pool_ref):
        i = pl.program_id(0)
        a = jnp.maximum(
            jnp.dot(h_ref[...], wa_ref[...], preferred_element_type=jnp.float32)
            + ba_ref[...], 0.0)
        y = jnp.maximum(
            jnp.dot(a, wb_ref[...], preferred_element_type=jnp.float32)
            + bb_ref[...], 0.0)
        out_ref[...] = y
        ids = ids_ref[0, 0, :]
        oh = (ids[:, None] == lax.broadcasted_iota(jnp.int32, (rblk, NB), 1)
              ).astype(jnp.bfloat16)
        part = lax.dot_general(oh, y.astype(jnp.bfloat16),
                               (((0,), (0,)), ((), ())),
                               preferred_element_type=jnp.float32)

        @pl.when(i == 0)
        def _():
            pool_ref[...] = jnp.zeros_like(pool_ref)

        pool_ref[...] += part

    return pl.pallas_call(
        body,
        grid=(grid,),
        in_specs=[
            pl.BlockSpec((rblk, D), lambda i: (i, 0)),
            pl.BlockSpec((D, D), lambda i: (0, 0)),
            pl.BlockSpec((1, D), lambda i: (0, 0)),
            pl.BlockSpec((D, D), lambda i: (0, 0)),
            pl.BlockSpec((1, D), lambda i: (0, 0)),
            pl.BlockSpec((1, 1, rblk), lambda i: (i, 0, 0)),
        ],
        out_specs=[
            pl.BlockSpec((rblk, D), lambda i: (i, 0)),
            pl.BlockSpec((NB, D), lambda i: (0, 0)),
        ],
        out_shape=[
            jax.ShapeDtypeStruct((n, D), jnp.float32),
            jax.ShapeDtypeStruct((NB, D), jnp.float32),
        ],
    )(h, wa, ba, wb, bb, ids3)


def _pool_only(x, ids3, rblk):
    n = x.shape[0]
    grid = n // rblk

    def body(x_ref, ids_ref, pool_ref):
        i = pl.program_id(0)
        ids = ids_ref[0, 0, :]
        oh = (ids[:, None] == lax.broadcasted_iota(jnp.int32, (rblk, NB), 1)
              ).astype(jnp.bfloat16)
        part = lax.dot_general(oh, x_ref[...].astype(jnp.bfloat16),
                               (((0,), (0,)), ((), ())),
                               preferred_element_type=jnp.float32)

        @pl.when(i == 0)
        def _():
            pool_ref[...] = jnp.zeros_like(pool_ref)

        pool_ref[...] += part

    return pl.pallas_call(
        body,
        grid=(grid,),
        in_specs=[
            pl.BlockSpec((rblk, D), lambda i: (i, 0)),
            pl.BlockSpec((1, 1, rblk), lambda i: (i, 0, 0)),
        ],
        out_specs=pl.BlockSpec((NB, D), lambda i: (0, 0)),
        out_shape=jax.ShapeDtypeStruct((NB, D), jnp.float32),
    )(x, ids3)


def _combine(p0, p1, p2, wd, bd):
    def body(p0_ref, p1_ref, p2_ref, wd_ref, bd_ref, out_ref):
        out_ref[...] = (
            jnp.dot(p0_ref[...], wd_ref[0:D, :], preferred_element_type=jnp.float32)
            + jnp.dot(p1_ref[...], wd_ref[D:2 * D, :], preferred_element_type=jnp.float32)
            + jnp.dot(p2_ref[...], wd_ref[2 * D:3 * D, :], preferred_element_type=jnp.float32)
            + bd_ref[...])

    return pl.pallas_call(
        body,
        out_shape=jax.ShapeDtypeStruct((NB, D), jnp.float32),
    )(p0, p1, p2, wd, bd)


def _pad_edges(e, e_pad, n_edges):
    fill = jnp.broadcast_to(jnp.array([[-1], [0]], jnp.int32), (2, e_pad - n_edges))
    return jnp.concatenate([e, fill], axis=1)


def _pad_ids(ids, npad, rblk):
    n = ids.shape[0]
    if npad > n:
        ids = jnp.concatenate([ids, jnp.full((npad - n,), NB, jnp.int32)])
    return ids.reshape(npad // rblk, 1, rblk)


def kernel(x, edge_index1, edge_index2, batch0, batch1, batch2,
           W1a, b1a, g1a, be1a, W1b, b1b, g1b, be1b,
           W2a, b2a, W2b, b2b, Wd, bd):
    s = 1.0 / jnp.sqrt(1.0 + EPS)
    W1a_f = W1a * (g1a * s)[None, :]
    b1a_f = (b1a * (g1a * s) + be1a)[None, :]
    W1b_f = W1b * (g1b * s)[None, :]
    b1b_f = (b1b * (g1b * s) + be1b)[None, :]

    e1p = _pad_edges(edge_index1, EPAD1, E1)
    e2p = _pad_edges(edge_index2, EPAD2, E2)

    agg1 = _agg1()(x, e1p)
    p0 = _pool_only(x, _pad_ids(batch0, 100000, 800), 800)
    x1p, p1 = _mlp_pool(agg1, W1a_f, b1a_f, W1b_f, b1b_f,
                        _pad_ids(batch1, NPAD1, 512), 512)
    agg2 = _agg2()(x1p, e2p)
    _, p2 = _mlp_pool(agg2, W2a, b2a[None, :], W2b, b2b[None, :],
                      _pad_ids(batch2, NPAD2, 512), 512)
    return _combine(p0, p1, p2, Wd, bd[None, :])


# speedup vs baseline: 3.9711x; 1.0034x over previous
import functools

import jax
import jax.numpy as jnp
from jax import lax
from jax.experimental import pallas as pl
from jax.experimental.pallas import tpu as pltpu
from jax.experimental.pallas import tpu_sc as plsc

N0, N1, N2 = 100000, 50000, 25000
E1, E2 = 500000, 250000
D = 128
NB = 1000
EPS = 1e-5

B = 8192
K = 64
KSH, KMSK = 6, 63
NTILES = 16
NSC = 2

EPAD1, NPAD1 = 524288, 51200
EPAD2, NPAD2 = 262144, 25600


def _make_sc_agg(n_pass, nbpt, ch):
    cpt = ch // NTILES
    nfull, rem = cpt // K, cpt % K
    cbuf = B + K + 16
    npad = n_pass * NSC * ch
    mesh = plsc.VectorSubcoreMesh(core_axis_name="c", subcore_axis_name="s",
                                  num_cores=NSC, num_subcores=NTILES)

    @functools.partial(
        pl.kernel,
        mesh=mesh,
        out_type=jax.ShapeDtypeStruct((npad, D), jnp.float32),
        scratch_types=[
            pltpu.VMEM((2, B), jnp.int32),
            pltpu.VMEM((cbuf,), jnp.int32),
            pltpu.VMEM((B // K + 1, K), jnp.int32),
            pltpu.VMEM((2, K, D), jnp.float32),
            pltpu.VMEM_SHARED((ch + 16, D), jnp.float32),
            pltpu.SemaphoreType.DMA((2,)),
        ],
        compiler_params=pltpu.CompilerParams(needs_layout_passes=False),
    )
    def agg(table, edges, out, ebuf, csrc, cdst, rows, acc, sems):
        c = lax.axis_index("c")
        s = lax.axis_index("s")
        r0 = s * cpt
        iota16 = lax.broadcasted_iota(jnp.int32, (16,), 0)

        for p in range(n_pass):
            lo = (p * NSC + c) * ch
            hi = lo + ch

            def zbody(i, _):
                rows[0, i // 8, pl.ds((i % 8) * 16, 16)] = jnp.zeros((16,), jnp.float32)
                return 0
            lax.fori_loop(0, K * 8, zbody, 0)
            for j in range(nfull):
                pltpu.sync_copy(rows.at[0], acc.at[pl.ds(r0 + j * K, K)])
            if rem:
                pltpu.sync_copy(rows.at[0, pl.ds(0, rem)],
                                acc.at[pl.ds(r0 + nfull * K, rem)])

            @pl.when(s == 0)
            def _():
                pltpu.sync_copy(rows.at[0, pl.ds(0, 16)], acc.at[pl.ds(ch, 16)])

            plsc.subcore_barrier()

            def fbody(it, _):
                off = (it * NTILES + s) * B
                pltpu.sync_copy(edges.at[:, pl.ds(off, B)], ebuf)

                def vbody(v, cnt):
                    dv = ebuf[0, pl.ds(v * 16, 16)]
                    sv = ebuf[1, pl.ds(v * 16, 16)]
                    m = (dv >= lo) & (dv < hi)
                    x = jnp.where(m, 1, 0)
                    for kk in (1, 2, 4, 8):
                        sh = x.at[jnp.maximum(iota16 - kk, 0)].get(
                            mode="promise_in_bounds")
                        x = x + jnp.where(iota16 >= kk, sh, 0)
                    pos = cnt + x - 1
                    plsc.store_scatter(csrc, [pos], sv, mask=m)
                    plsc.store_scatter(
                        cdst, [lax.shift_right_logical(pos, KSH),
                               lax.bitwise_and(pos, KMSK)], dv - lo, mask=m)
                    return cnt + x[15]

                cnt = lax.fori_loop(0, B // 16, vbody, jnp.int32(0))

                for j in range(K // 16):
                    ppos = cnt + j * 16 + iota16
                    plsc.store_scatter(csrc, [ppos], jnp.zeros((16,), jnp.int32))
                    plsc.store_scatter(
                        cdst, [lax.shift_right_logical(ppos, KSH),
                               lax.bitwise_and(ppos, KMSK)],
                        jnp.full((16,), ch, jnp.int32))
                nblk = (cnt + K - 1) // K

                @pl.when(nblk > 0)
                def _():
                    pltpu.async_copy(table.at[csrc.at[pl.ds(0, K)]],
                                     rows.at[0], sems.at[0])

                def gbody(i, _):
                    par = lax.rem(i, 2)
                    nxt = lax.rem(i + 1, 2)

                    @pl.when(i + 1 < nblk)
                    def _():
                        pltpu.async_copy(
                            table.at[csrc.at[pl.ds((i + 1) * K, K)]],
                            rows.at[nxt], sems.at[nxt])

                    pltpu.make_async_copy(
                        table.at[csrc.at[pl.ds(i * K, K)]],
                        rows.at[par], sems.at[par]).wait()
                    pltpu.sync_copy(rows.at[par], acc.at[cdst.at[i]], add=True)
                    return 0

                lax.fori_loop(0, nblk, gbody, 0)
                return 0

            lax.fori_loop(0, nbpt, fbody, jnp.int32(0))
            plsc.subcore_barrier()

            pltpu.sync_copy(acc.at[pl.ds(r0, cpt)], out.at[pl.ds(lo + r0, cpt)])

    return agg


@functools.lru_cache(maxsize=None)
def _agg1():
    return _make_sc_agg(n_pass=4, nbpt=4, ch=6400)


@functools.lru_cache(maxsize=None)
def _agg2():
    return _make_sc_agg(n_pass=2, nbpt=2, ch=6400)


def _mlp_pool(h, wa, ba, wb, bb, ids3, rblk):
    n = h.shape[0]
    grid = n // rblk

    def body(h_ref, wa_ref, ba_ref, wb_ref, bb_ref, ids_ref, out_ref, pool_ref):
        i = pl.program_id(0)
        a = jnp.maximum(
            jnp.dot(h_ref[...], wa_ref[...], preferred_element_type=jnp.float32)
            + ba_ref[...], 0.0)
        y = jnp.maximum(
            jnp.dot(a, wb_ref[...], preferred_element_type=jnp.float32)
            + bb_ref[...], 0.0)
        out_ref[...] = y
        ids = ids_ref[0, 0, :]
        oh = (ids[:, None] == lax.broadcasted_iota(jnp.int32, (rblk, NB), 1)
              ).astype(jnp.bfloat16)
        part = lax.dot_general(oh, y.astype(jnp.bfloat16),
                               (((0,), (0,)), ((), ())),
                               preferred_element_type=jnp.float32)

        @pl.when(i == 0)
        def _():
            pool_ref[...] = jnp.zeros_like(pool_ref)

        pool_ref[...] += part

    return pl.pallas_call(
        body,
        grid=(grid,),
        in_specs=[
            pl.BlockSpec((rblk, D), lambda i: (i, 0)),
            pl.BlockSpec((D, D), lambda i: (0, 0)),
            pl.BlockSpec((1, D), lambda i: (0, 0)),
            pl.BlockSpec((D, D), lambda i: (0, 0)),
            pl.BlockSpec((1, D), lambda i: (0, 0)),
            pl.BlockSpec((1, 1, rblk), lambda i: (i, 0, 0)),
        ],
        out_specs=[
            pl.BlockSpec((rblk, D), lambda i: (i, 0)),
            pl.BlockSpec((NB, D), lambda i: (0, 0)),
        ],
        out_shape=[
            jax.ShapeDtypeStruct((n, D), jnp.float32),
            jax.ShapeDtypeStruct((NB, D), jnp.float32),
        ],
    )(h, wa, ba, wb, bb, ids3)


def _pool_only(x, ids3, rblk):
    n = x.shape[0]
    grid = n // rblk

    def body(x_ref, ids_ref, pool_ref):
        i = pl.program_id(0)
        ids = ids_ref[0, 0, :]
        oh = (ids[:, None] == lax.broadcasted_iota(jnp.int32, (rblk, NB), 1)
              ).astype(jnp.bfloat16)
        part = lax.dot_general(oh, x_ref[...].astype(jnp.bfloat16),
                               (((0,), (0,)), ((), ())),
                               preferred_element_type=jnp.float32)

        @pl.when(i == 0)
        def _():
            pool_ref[...] = jnp.zeros_like(pool_ref)

        pool_ref[...] += part

    return pl.pallas_call(
        body,
        grid=(grid,),
        in_specs=[
            pl.BlockSpec((rblk, D), lambda i: (i, 0)),
            pl.BlockSpec((1, 1, rblk), lambda i: (i, 0, 0)),
        ],
        out_specs=pl.BlockSpec((NB, D), lambda i: (0, 0)),
        out_shape=jax.ShapeDtypeStruct((NB, D), jnp.float32),
    )(x, ids3)


def _combine(p0, p1, p2, wd, bd):
    def body(p0_ref, p1_ref, p2_ref, wd_ref, bd_ref, out_ref):
        out_ref[...] = (
            jnp.dot(p0_ref[...], wd_ref[0:D, :], preferred_element_type=jnp.float32)
            + jnp.dot(p1_ref[...], wd_ref[D:2 * D, :], preferred_element_type=jnp.float32)
            + jnp.dot(p2_ref[...], wd_ref[2 * D:3 * D, :], preferred_element_type=jnp.float32)
            + bd_ref[...])

    return pl.pallas_call(
        body,
        out_shape=jax.ShapeDtypeStruct((NB, D), jnp.float32),
    )(p0, p1, p2, wd, bd)


def _pad_edges(e, e_pad, n_edges):
    fill = jnp.broadcast_to(jnp.array([[-1], [0]], jnp.int32), (2, e_pad - n_edges))
    return jnp.concatenate([e, fill], axis=1)


def _pad_ids(ids, npad, rblk):
    n = ids.shape[0]
    if npad > n:
        ids = jnp.concatenate([ids, jnp.full((npad - n,), NB, jnp.int32)])
    return ids.reshape(npad // rblk, 1, rblk)


def kernel(x, edge_index1, edge_index2, batch0, batch1, batch2,
           W1a, b1a, g1a, be1a, W1b, b1b, g1b, be1b,
           W2a, b2a, W2b, b2b, Wd, bd):
    s = 1.0 / jnp.sqrt(1.0 + EPS)
    W1a_f = W1a * (g1a * s)[None, :]
    b1a_f = (b1a * (g1a * s) + be1a)[None, :]
    W1b_f = W1b * (g1b * s)[None, :]
    b1b_f = (b1b * (g1b * s) + be1b)[None, :]

    e1p = _pad_edges(edge_index1, EPAD1, E1)
    e2p = _pad_edges(edge_index2, EPAD2, E2)

    agg1 = _agg1()(x, e1p)
    p0 = _pool_only(x, _pad_ids(batch0, 100000, 800), 800)
    x1p, p1 = _mlp_pool(agg1, W1a_f, b1a_f, W1b_f, b1b_f,
                        _pad_ids(batch1, NPAD1, 512), 512)
    agg2 = _agg2()(x1p, e2p)
    _, p2 = _mlp_pool(agg2, W2a, b2a[None, :], W2b, b2b[None, :],
                      _pad_ids(batch2, NPAD2, 512), 512)
    return _combine(p0, p1, p2, Wd, bd[None, :])
